# baseline (speedup 1.0000x reference)
import functools

import jax
import jax.numpy as jnp
from jax import lax
from jax.experimental import pallas as pl
from jax.experimental.pallas import tpu as pltpu
from jax.experimental.pallas import tpu_sc as plsc

_NUM_CORES = 2
_NUM_SUBCORES = 16
_NUM_WORKERS = _NUM_CORES * _NUM_SUBCORES

_VOCAB_TILE = 1024


@functools.cache
def _make_sc_gather(V, D, B, idx_dtype):
    assert B % (8 * _NUM_WORKERS) == 0
    b_per_w = B // _NUM_WORKERS
    mesh = plsc.VectorSubcoreMesh(core_axis_name="c", subcore_axis_name="s")

    @functools.partial(
        pl.kernel,
        mesh=mesh,
        out_type=jax.ShapeDtypeStruct((B, D), jnp.float32),
        scratch_types=[
            pltpu.VMEM((b_per_w,), jnp.int32),
            pltpu.VMEM((b_per_w, D), jnp.float32),
            pltpu.SemaphoreType.DMA,
        ],
        compiler_params=pltpu.CompilerParams(use_tc_tiling_on_sc=False),
    )
    def gather(table_hbm, idx_hbm, out_hbm, idx_v, rows_v, sem):
        wid = lax.axis_index("s") * _NUM_CORES + lax.axis_index("c")
        base = wid * b_per_w
        pltpu.sync_copy(idx_hbm.at[pl.ds(base, b_per_w)], idx_v)
        pltpu.async_copy(table_hbm.at[idx_v], rows_v, sem).wait()
        pltpu.sync_copy(rows_v, out_hbm.at[pl.ds(base, b_per_w)])

    return gather


def _proj_body(emb_ref, w_ref, b_ref, out_ref):
    out_ref[...] = lax.dot_general(
        emb_ref[...],
        w_ref[...],
        dimension_numbers=(((1,), (1,)), ((), ())),
        preferred_element_type=jnp.float32,
    ) + b_ref[...]


@functools.cache
def _make_projection(B, E, V):
    nt = pl.cdiv(V, _VOCAB_TILE)
    return pl.pallas_call(
        _proj_body,
        grid=(nt,),
        in_specs=[
            pl.BlockSpec((B, E), lambda i: (0, 0)),
            pl.BlockSpec((_VOCAB_TILE, E), lambda i: (i, 0)),
            pl.BlockSpec((1, _VOCAB_TILE), lambda i: (0, i)),
        ],
        out_specs=pl.BlockSpec((B, _VOCAB_TILE), lambda i: (0, i)),
        out_shape=jax.ShapeDtypeStruct((B, V), jnp.float32),
        compiler_params=pltpu.CompilerParams(
            dimension_semantics=("arbitrary",),
        ),
    )


def kernel(center_idx, emb_table, W, b):
    idx = center_idx.astype(jnp.int32)
    V, E = emb_table.shape
    B = idx.shape[0]
    emb = _make_sc_gather(V, E, B, idx.dtype)(emb_table, idx)
    return _make_projection(B, E, V)(emb, W, b.reshape(1, V))


# speedup vs baseline: 1.0381x; 1.0381x over previous
import functools

import jax
import jax.numpy as jnp
from jax import lax
from jax.experimental import pallas as pl
from jax.experimental.pallas import tpu as pltpu
from jax.experimental.pallas import tpu_sc as plsc

_NUM_CORES = 2
_NUM_SUBCORES = 16
_NUM_WORKERS = _NUM_CORES * _NUM_SUBCORES

_VOCAB_TILE = 1024


@functools.cache
def _make_sc_gather(V, D, B, idx_dtype):
    assert B % (8 * _NUM_WORKERS) == 0
    b_per_w = B // _NUM_WORKERS
    mesh = plsc.VectorSubcoreMesh(core_axis_name="c", subcore_axis_name="s")

    @functools.partial(
        pl.kernel,
        mesh=mesh,
        out_type=jax.ShapeDtypeStruct((B, D), jnp.float32),
        scratch_types=[
            pltpu.VMEM((b_per_w,), jnp.int32),
            pltpu.VMEM((b_per_w, D), jnp.float32),
            pltpu.SemaphoreType.DMA,
        ],
        compiler_params=pltpu.CompilerParams(use_tc_tiling_on_sc=False),
    )
    def gather(table_hbm, idx_hbm, out_hbm, idx_v, rows_v, sem):
        wid = lax.axis_index("s") * _NUM_CORES + lax.axis_index("c")
        base = wid * b_per_w
        pltpu.sync_copy(idx_hbm.at[pl.ds(base, b_per_w)], idx_v)
        pltpu.async_copy(table_hbm.at[idx_v], rows_v, sem).wait()
        pltpu.sync_copy(rows_v, out_hbm.at[pl.ds(base, b_per_w)])

    return gather


def _proj_body(emb_ref, w_ref, b_ref, out_ref):
    out_ref[...] = lax.dot_general(
        emb_ref[...],
        w_ref[...],
        dimension_numbers=(((1,), (1,)), ((), ())),
        preferred_element_type=jnp.float32,
    ) + b_ref[...]


@functools.cache
def _make_projection(B, E, V):
    nt = pl.cdiv(V, _VOCAB_TILE)
    return pl.pallas_call(
        _proj_body,
        grid=(nt,),
        in_specs=[
            pl.BlockSpec((B, E), lambda i: (0, 0)),
            pl.BlockSpec((_VOCAB_TILE, E), lambda i: (i, 0)),
            pl.BlockSpec((1, _VOCAB_TILE), lambda i: (0, i)),
        ],
        out_specs=pl.BlockSpec((B, _VOCAB_TILE), lambda i: (0, i)),
        out_shape=jax.ShapeDtypeStruct((B, V), jnp.float32),
        compiler_params=pltpu.CompilerParams(
            dimension_semantics=("arbitrary",),
        ),
    )


def kernel(center_idx, emb_table, W, b):
    idx = center_idx.astype(jnp.int32)
    V, E = emb_table.shape
    B = idx.shape[0]
    emb = jnp.take(emb_table, idx, axis=0)
    return _make_projection(B, E, V)(emb, W, b.reshape(1, V))


# speedup vs baseline: 1.0751x; 1.0356x over previous
import functools

import jax
import jax.numpy as jnp
from jax import lax
from jax.experimental import pallas as pl
from jax.experimental.pallas import tpu as pltpu
from jax.experimental.pallas import tpu_sc as plsc

_NUM_CORES = 2
_NUM_SUBCORES = 16
_NUM_WORKERS = _NUM_CORES * _NUM_SUBCORES

_VOCAB_TILE = 2048


@functools.cache
def _make_sc_gather(V, D, B, idx_dtype):
    assert B % (8 * _NUM_WORKERS) == 0
    b_per_w = B // _NUM_WORKERS
    mesh = plsc.VectorSubcoreMesh(core_axis_name="c", subcore_axis_name="s")

    @functools.partial(
        pl.kernel,
        mesh=mesh,
        out_type=jax.ShapeDtypeStruct((B, D), jnp.float32),
        scratch_types=[
            pltpu.VMEM((b_per_w,), jnp.int32),
            pltpu.VMEM((b_per_w, D), jnp.float32),
            pltpu.SemaphoreType.DMA,
        ],
        compiler_params=pltpu.CompilerParams(use_tc_tiling_on_sc=False),
    )
    def gather(table_hbm, idx_hbm, out_hbm, idx_v, rows_v, sem):
        wid = lax.axis_index("s") * _NUM_CORES + lax.axis_index("c")
        base = wid * b_per_w
        pltpu.sync_copy(idx_hbm.at[pl.ds(base, b_per_w)], idx_v)
        pltpu.async_copy(table_hbm.at[idx_v], rows_v, sem).wait()
        pltpu.sync_copy(rows_v, out_hbm.at[pl.ds(base, b_per_w)])

    return gather


def _proj_body(emb_ref, w_ref, b_ref, out_ref):
    out_ref[...] = lax.dot_general(
        emb_ref[...],
        w_ref[...],
        dimension_numbers=(((1,), (1,)), ((), ())),
        preferred_element_type=jnp.float32,
    ) + b_ref[...]


@functools.cache
def _make_projection(B, E, V):
    nt = pl.cdiv(V, _VOCAB_TILE)
    return pl.pallas_call(
        _proj_body,
        grid=(nt,),
        in_specs=[
            pl.BlockSpec((B, E), lambda i: (0, 0)),
            pl.BlockSpec((_VOCAB_TILE, E), lambda i: (i, 0)),
            pl.BlockSpec((1, _VOCAB_TILE), lambda i: (0, i)),
        ],
        out_specs=pl.BlockSpec((B, _VOCAB_TILE), lambda i: (0, i)),
        out_shape=jax.ShapeDtypeStruct((B, V), jnp.float32),
        compiler_params=pltpu.CompilerParams(
            dimension_semantics=("arbitrary",),
        ),
    )


def kernel(center_idx, emb_table, W, b):
    idx = center_idx.astype(jnp.int32)
    V, E = emb_table.shape
    B = idx.shape[0]
    emb = jnp.take(emb_table, idx, axis=0)
    return _make_projection(B, E, V)(emb, W, b.reshape(1, V))


# speedup vs baseline: 1.1575x; 1.0767x over previous
import functools

import jax
import jax.numpy as jnp
from jax import lax
from jax.experimental import pallas as pl
from jax.experimental.pallas import tpu as pltpu
from jax.experimental.pallas import tpu_sc as plsc

_NUM_CORES = 2
_NUM_SUBCORES = 16
_NUM_WORKERS = _NUM_CORES * _NUM_SUBCORES

_VOCAB_TILE = 2048


@functools.cache
def _make_sc_gather(V, D, B, idx_dtype):
    assert B % (8 * _NUM_WORKERS) == 0
    b_per_w = B // _NUM_WORKERS
    mesh = plsc.VectorSubcoreMesh(core_axis_name="c", subcore_axis_name="s")

    @functools.partial(
        pl.kernel,
        mesh=mesh,
        out_type=jax.ShapeDtypeStruct((B, D), jnp.float32),
        scratch_types=[
            pltpu.VMEM((b_per_w,), jnp.int32),
            pltpu.VMEM((b_per_w, D), jnp.float32),
            pltpu.SemaphoreType.DMA,
        ],
        compiler_params=pltpu.CompilerParams(use_tc_tiling_on_sc=False),
    )
    def gather(table_hbm, idx_hbm, out_hbm, idx_v, rows_v, sem):
        wid = lax.axis_index("s") * _NUM_CORES + lax.axis_index("c")
        base = wid * b_per_w
        pltpu.sync_copy(idx_hbm.at[pl.ds(base, b_per_w)], idx_v)
        pltpu.async_copy(table_hbm.at[idx_v], rows_v, sem).wait()
        pltpu.sync_copy(rows_v, out_hbm.at[pl.ds(base, b_per_w)])

    return gather


def _proj_body(embt_ref, wt_ref, b_ref, out_ref):
    out_ref[...] = lax.dot_general(
        embt_ref[...],
        wt_ref[...],
        dimension_numbers=(((0,), (0,)), ((), ())),
        preferred_element_type=jnp.float32,
    ) + b_ref[...]


@functools.cache
def _make_projection(B, E, V):
    nt = pl.cdiv(V, _VOCAB_TILE)
    return pl.pallas_call(
        _proj_body,
        grid=(nt,),
        in_specs=[
            pl.BlockSpec((E, B), lambda i: (0, 0)),
            pl.BlockSpec((E, _VOCAB_TILE), lambda i: (0, i)),
            pl.BlockSpec((1, _VOCAB_TILE), lambda i: (0, i)),
        ],
        out_specs=pl.BlockSpec((B, _VOCAB_TILE), lambda i: (0, i)),
        out_shape=jax.ShapeDtypeStruct((B, V), jnp.float32),
        compiler_params=pltpu.CompilerParams(
            dimension_semantics=("arbitrary",),
        ),
    )


def kernel(center_idx, emb_table, W, b):
    idx = center_idx.astype(jnp.int32)
    V, E = emb_table.shape
    B = idx.shape[0]
    emb = jnp.take(emb_table, idx, axis=0)
    return _make_projection(B, E, V)(emb.T, W.T, b.reshape(1, V))


# speedup vs baseline: 1.1640x; 1.0056x over previous
import functools

import jax
import jax.numpy as jnp
from jax import lax
from jax.experimental import pallas as pl
from jax.experimental.pallas import tpu as pltpu
from jax.experimental.pallas import tpu_sc as plsc

_NUM_CORES = 2
_NUM_SUBCORES = 16
_NUM_WORKERS = _NUM_CORES * _NUM_SUBCORES

_VOCAB_TILE = 1024


@functools.cache
def _make_sc_gather(V, D, B, idx_dtype):
    assert B % (8 * _NUM_WORKERS) == 0
    b_per_w = B // _NUM_WORKERS
    mesh = plsc.VectorSubcoreMesh(core_axis_name="c", subcore_axis_name="s")

    @functools.partial(
        pl.kernel,
        mesh=mesh,
        out_type=jax.ShapeDtypeStruct((B, D), jnp.float32),
        scratch_types=[
            pltpu.VMEM((b_per_w,), jnp.int32),
            pltpu.VMEM((b_per_w, D), jnp.float32),
            pltpu.SemaphoreType.DMA,
        ],
        compiler_params=pltpu.CompilerParams(use_tc_tiling_on_sc=False),
    )
    def gather(table_hbm, idx_hbm, out_hbm, idx_v, rows_v, sem):
        wid = lax.axis_index("s") * _NUM_CORES + lax.axis_index("c")
        base = wid * b_per_w
        pltpu.sync_copy(idx_hbm.at[pl.ds(base, b_per_w)], idx_v)
        pltpu.async_copy(table_hbm.at[idx_v], rows_v, sem).wait()
        pltpu.sync_copy(rows_v, out_hbm.at[pl.ds(base, b_per_w)])

    return gather


_BATCH_TILE = 32


def _proj_body(emb_ref, wt_ref, b_ref, out_ref):
    out_ref[...] = lax.dot_general(
        emb_ref[...],
        wt_ref[...],
        dimension_numbers=(((1,), (0,)), ((), ())),
        preferred_element_type=jnp.float32,
    ) + b_ref[...]


@functools.cache
def _make_projection(B, E, V):
    nt = B // _BATCH_TILE
    return pl.pallas_call(
        _proj_body,
        grid=(nt,),
        in_specs=[
            pl.BlockSpec((_BATCH_TILE, E), lambda i: (i, 0)),
            pl.BlockSpec((E, V), lambda i: (0, 0)),
            pl.BlockSpec((1, V), lambda i: (0, 0)),
        ],
        out_specs=pl.BlockSpec((_BATCH_TILE, V), lambda i: (i, 0)),
        out_shape=jax.ShapeDtypeStruct((B, V), jnp.float32),
        compiler_params=pltpu.CompilerParams(
            dimension_semantics=("arbitrary",),
            vmem_limit_bytes=100 * 1024 * 1024,
        ),
    )


def kernel(center_idx, emb_table, W, b):
    idx = center_idx.astype(jnp.int32)
    V, E = emb_table.shape
    B = idx.shape[0]
    emb = jnp.take(emb_table, idx, axis=0)
    return _make_projection(B, E, V)(emb, W.T, b.reshape(1, V))
